# Initial kernel scaffold; baseline (speedup 1.0000x reference)
import jax
import jax.numpy as jnp
from jax.experimental import pallas as pl


def kernel(x, edge_index, W1, b1, W2, b2):
    raise NotImplementedError("write your pallas kernel here")


# speedup vs baseline: 14.6082x; 14.6082x over previous
import functools

import jax
import jax.numpy as jnp
from jax import lax
from jax.experimental import pallas as pl
from jax.experimental.pallas import tpu as pltpu
from jax.experimental.pallas import tpu_sc as plsc

NC = 2
NS = 16
L = 16
NW = NC * NS

K_EDGE = 80
BM = 1000


def _mesh():
    return plsc.VectorSubcoreMesh(
        core_axis_name="c", subcore_axis_name="s", num_cores=NC, num_subcores=NS
    )


@functools.lru_cache(None)
def _make_deg_kernel(E, n_pad, K):
    nchunks = E // (NW * K)
    epw = nchunks * K
    rpt = n_pad // NS

    @functools.partial(
        pl.kernel,
        out_type=jax.ShapeDtypeStruct((NC, n_pad, L), jnp.float32),
        mesh=_mesh(),
        scratch_types=[
            pltpu.VMEM((1, K), jnp.int32),
            pltpu.VMEM((K, L), jnp.float32),
            pltpu.VMEM((16, L), jnp.float32),
            pltpu.VMEM_SHARED((n_pad, L), jnp.float32),
        ],
        compiler_params=pltpu.CompilerParams(use_tc_tiling_on_sc=False),
    )
    def deg_kernel(dst_hbm, out_hbm, idx_v, ones_v, zero_v, acc_sh):
        cid = lax.axis_index("c")
        sid = lax.axis_index("s")
        wid = sid * NC + cid

        def fill_ones(r, _):
            ones_v[r, :] = jnp.full((L,), 1.0, jnp.float32)
            return 0

        lax.fori_loop(0, K, fill_ones, 0)

        def fill_zero(r, _):
            zero_v[r, :] = jnp.zeros((L,), jnp.float32)
            return 0

        lax.fori_loop(0, 16, fill_zero, 0)

        base = sid * rpt

        def zero_acc(i, _):
            pltpu.sync_copy(zero_v, acc_sh.at[pl.ds(base + i * 16, 16)])
            return 0

        lax.fori_loop(0, rpt // 16, zero_acc, 0)
        plsc.subcore_barrier()

        ebase = wid * epw

        def body(ci, _):
            pltpu.sync_copy(dst_hbm.at[pl.ds(ebase + ci * K, K)], idx_v.at[0])
            pltpu.sync_copy(ones_v, acc_sh.at[idx_v.at[0]], add=True)
            return 0

        lax.fori_loop(0, nchunks, body, 0)
        plsc.subcore_barrier()
        pltpu.sync_copy(
            acc_sh.at[pl.ds(base, rpt)], out_hbm.at[cid, pl.ds(base, rpt)]
        )

    return deg_kernel


@functools.lru_cache(None)
def _make_scatter_kernel(width, E, n_pad, K):
    nchunks = E // (NW * K)
    epw = nchunks * K
    rpt = n_pad // NS

    @functools.partial(
        pl.kernel,
        out_type=jax.ShapeDtypeStruct((NC, n_pad, width), jnp.float32),
        mesh=_mesh(),
        scratch_types=[
            pltpu.VMEM((1, K), jnp.int32),
            pltpu.VMEM((1, K), jnp.int32),
            pltpu.VMEM((K, width), jnp.float32),
            pltpu.VMEM((16, width), jnp.float32),
            pltpu.VMEM_SHARED((n_pad, width), jnp.float32),
            pltpu.SemaphoreType.DMA,
        ],
        compiler_params=pltpu.CompilerParams(use_tc_tiling_on_sc=False),
    )
    def scatter_kernel(
        g_hbm, src_hbm, dst_hbm, out_hbm, src_v, dst_v, rows_v, zero_v, acc_sh, sem
    ):
        cid = lax.axis_index("c")
        sid = lax.axis_index("s")
        wid = sid * NC + cid

        def fill_zero(r, _):
            for c in range(width // L):
                zero_v[r, pl.ds(c * L, L)] = jnp.zeros((L,), jnp.float32)
            return 0

        lax.fori_loop(0, 16, fill_zero, 0)

        base = sid * rpt

        def zero_acc(i, _):
            pltpu.sync_copy(zero_v, acc_sh.at[pl.ds(base + i * 16, 16)])
            return 0

        lax.fori_loop(0, rpt // 16, zero_acc, 0)
        plsc.subcore_barrier()

        ebase = wid * epw

        def body(ci, _):
            off = ebase + ci * K
            pltpu.sync_copy(src_hbm.at[pl.ds(off, K)], src_v.at[0])
            pltpu.sync_copy(dst_hbm.at[pl.ds(off, K)], dst_v.at[0])
            pltpu.async_copy(g_hbm.at[src_v.at[0]], rows_v, sem).wait()
            pltpu.sync_copy(rows_v, acc_sh.at[dst_v.at[0]], add=True)
            return 0

        lax.fori_loop(0, nchunks, body, 0)
        plsc.subcore_barrier()
        pltpu.sync_copy(
            acc_sh.at[pl.ds(base, rpt)], out_hbm.at[cid, pl.ds(base, rpt)]
        )

    return scatter_kernel


def _tc1_body(x_ref, w_ref, d0_ref, d1_ref, g_ref, dinv_ref):
    deg = d0_ref[...] + d1_ref[...] + 1.0
    dinv = lax.rsqrt(jnp.maximum(deg, 1.0))
    h = jnp.dot(x_ref[...], w_ref[...], preferred_element_type=jnp.float32)
    g_ref[...] = h * dinv
    dinv_ref[...] = dinv


def _tc2_body(s0_ref, s1_ref, g_ref, dinv_ref, b_ref, w_ref, out_ref):
    dinv = dinv_ref[...]
    h = dinv * (s0_ref[...] + s1_ref[...] + g_ref[...]) + b_ref[...]
    h = jnp.maximum(h, 0.0)
    out_ref[...] = (
        jnp.dot(h, w_ref[...], preferred_element_type=jnp.float32) * dinv
    )


def _tc3_body(s0_ref, s1_ref, g_ref, dinv_ref, b_ref, out_ref):
    out_ref[...] = (
        dinv_ref[...] * (s0_ref[...] + s1_ref[...] + g_ref[...]) + b_ref[...]
    )


def kernel(x, edge_index, W1, b1, W2, b2):
    N, D = x.shape
    H = W1.shape[1]
    C = W2.shape[1]
    E = edge_index.shape[1]

    n_pad = -(-N // (NS * 16)) * (NS * 16)
    Cp = -(-C // L) * L

    src = edge_index[0]
    dst = edge_index[1]
    e_pad = -(-E // (NW * K_EDGE)) * (NW * K_EDGE) - E
    if e_pad:
        src = jnp.concatenate([src, jnp.zeros((e_pad,), src.dtype)])
        dst = jnp.concatenate([dst, jnp.full((e_pad,), n_pad - 1, dst.dtype)])
    Et = E + e_pad

    degp = _make_deg_kernel(Et, n_pad, K_EDGE)(dst)
    d0 = degp[0, :N, 0:1]
    d1 = degp[1, :N, 0:1]

    grid = (N // BM,)
    g1, dinv = pl.pallas_call(
        _tc1_body,
        grid=grid,
        in_specs=[
            pl.BlockSpec((BM, D), lambda i: (i, 0)),
            pl.BlockSpec((D, H), lambda i: (0, 0)),
            pl.BlockSpec((BM, 1), lambda i: (i, 0)),
            pl.BlockSpec((BM, 1), lambda i: (i, 0)),
        ],
        out_specs=[
            pl.BlockSpec((BM, H), lambda i: (i, 0)),
            pl.BlockSpec((BM, 1), lambda i: (i, 0)),
        ],
        out_shape=[
            jax.ShapeDtypeStruct((N, H), jnp.float32),
            jax.ShapeDtypeStruct((N, 1), jnp.float32),
        ],
    )(x, W1, d0, d1)

    s1 = _make_scatter_kernel(H, Et, n_pad, K_EDGE)(g1, src, dst)

    W2p = jnp.pad(W2, ((0, 0), (0, Cp - C)))
    b1r = b1.reshape(1, H)
    g2 = pl.pallas_call(
        _tc2_body,
        grid=grid,
        in_specs=[
            pl.BlockSpec((BM, H), lambda i: (i, 0)),
            pl.BlockSpec((BM, H), lambda i: (i, 0)),
            pl.BlockSpec((BM, H), lambda i: (i, 0)),
            pl.BlockSpec((BM, 1), lambda i: (i, 0)),
            pl.BlockSpec((1, H), lambda i: (0, 0)),
            pl.BlockSpec((H, Cp), lambda i: (0, 0)),
        ],
        out_specs=pl.BlockSpec((BM, Cp), lambda i: (i, 0)),
        out_shape=jax.ShapeDtypeStruct((N, Cp), jnp.float32),
    )(s1[0, :N], s1[1, :N], g1, dinv, b1r, W2p)

    s2 = _make_scatter_kernel(Cp, Et, n_pad, K_EDGE)(g2, src, dst)

    b2r = jnp.pad(b2, (0, Cp - C)).reshape(1, Cp)
    out = pl.pallas_call(
        _tc3_body,
        grid=grid,
        in_specs=[
            pl.BlockSpec((BM, Cp), lambda i: (i, 0)),
            pl.BlockSpec((BM, Cp), lambda i: (i, 0)),
            pl.BlockSpec((BM, Cp), lambda i: (i, 0)),
            pl.BlockSpec((BM, 1), lambda i: (i, 0)),
            pl.BlockSpec((1, Cp), lambda i: (0, 0)),
        ],
        out_specs=pl.BlockSpec((BM, Cp), lambda i: (i, 0)),
        out_shape=jax.ShapeDtypeStruct((N, Cp), jnp.float32),
    )(s2[0, :N], s2[1, :N], g2, dinv, b2r)

    return out[:, :C]


# speedup vs baseline: 18.1533x; 1.2427x over previous
import functools

import jax
import jax.numpy as jnp
from jax import lax
from jax.experimental import pallas as pl
from jax.experimental.pallas import tpu as pltpu
from jax.experimental.pallas import tpu_sc as plsc

NC = 2
NS = 16
L = 16
NW = NC * NS

K_EDGE = 128
NB = 2
ZR = 64
BM = 1000


def _mesh():
    return plsc.VectorSubcoreMesh(
        core_axis_name="c", subcore_axis_name="s", num_cores=NC, num_subcores=NS
    )


@functools.lru_cache(None)
def _make_deg_kernel(nchunks, n_pad, K):
    rpt = n_pad // NS
    QD = 8

    @functools.partial(
        pl.kernel,
        out_type=jax.ShapeDtypeStruct((NC, n_pad, L), jnp.float32),
        mesh=_mesh(),
        scratch_types=[
            pltpu.VMEM((nchunks, K), jnp.int32),
            pltpu.VMEM((K, L), jnp.float32),
            pltpu.VMEM((ZR, L), jnp.float32),
            pltpu.VMEM_SHARED((n_pad, L), jnp.float32),
            pltpu.SemaphoreType.DMA,
        ],
        compiler_params=pltpu.CompilerParams(use_tc_tiling_on_sc=False),
    )
    def deg_kernel(dst_hbm, out_hbm, dst_t, ones_v, zero_v, acc_sh, ssc):
        cid = lax.axis_index("c")
        sid = lax.axis_index("s")
        wid = sid * NC + cid

        pltpu.sync_copy(dst_hbm.at[pl.ds(wid * nchunks, nchunks)], dst_t)

        def fill_ones(r, _):
            ones_v[r, :] = jnp.full((L,), 1.0, jnp.float32)
            return 0

        lax.fori_loop(0, K, fill_ones, 0)

        def fill_zero(r, _):
            zero_v[r, :] = jnp.zeros((L,), jnp.float32)
            return 0

        lax.fori_loop(0, ZR, fill_zero, 0)

        base = sid * rpt

        def zero_acc(i, _):
            pltpu.sync_copy(zero_v, acc_sh.at[pl.ds(base + i * ZR, ZR)])
            return 0

        lax.fori_loop(0, rpt // ZR, zero_acc, 0)
        plsc.subcore_barrier()

        for j in range(QD):
            pltpu.async_copy(ones_v, acc_sh.at[dst_t.at[j]], ssc, add=True)

        def body(ci, _):
            pltpu.make_async_copy(ones_v, acc_sh.at[dst_t.at[ci]], ssc).wait()
            pltpu.async_copy(ones_v, acc_sh.at[dst_t.at[ci + QD]], ssc, add=True)
            return 0

        lax.fori_loop(0, nchunks - QD, body, 0)

        def drain(j, _):
            pltpu.make_async_copy(
                ones_v, acc_sh.at[dst_t.at[nchunks - QD + j]], ssc
            ).wait()
            return 0

        lax.fori_loop(0, QD, drain, 0)
        plsc.subcore_barrier()
        pltpu.sync_copy(
            acc_sh.at[pl.ds(base, rpt)], out_hbm.at[cid, pl.ds(base, rpt)]
        )

    return deg_kernel


@functools.lru_cache(None)
def _make_scatter_kernel(width, nchunks, n_pad, K):
    rpt = n_pad // NS
    ngroups = nchunks // NB

    @functools.partial(
        pl.kernel,
        out_type=jax.ShapeDtypeStruct((NC, n_pad, width), jnp.float32),
        mesh=_mesh(),
        scratch_types=[
            pltpu.VMEM((nchunks, K), jnp.int32),
            pltpu.VMEM((nchunks, K), jnp.int32),
            pltpu.VMEM((2 * NB, K, width), jnp.float32),
            pltpu.VMEM((ZR, width), jnp.float32),
            pltpu.VMEM_SHARED((n_pad, width), jnp.float32),
        ]
        + [pltpu.SemaphoreType.DMA] * (2 * NB)
        + [pltpu.SemaphoreType.DMA],
        compiler_params=pltpu.CompilerParams(use_tc_tiling_on_sc=False),
    )
    def scatter_kernel(
        g_hbm, src_hbm, dst_hbm, out_hbm, src_t, dst_t, rows_v, zero_v, acc_sh, *sems
    ):
        sg = sems[: 2 * NB]
        ssc = sems[2 * NB]
        cid = lax.axis_index("c")
        sid = lax.axis_index("s")
        wid = sid * NC + cid

        pltpu.sync_copy(src_hbm.at[pl.ds(wid * nchunks, nchunks)], src_t)
        pltpu.sync_copy(dst_hbm.at[pl.ds(wid * nchunks, nchunks)], dst_t)

        def gather(ci, b):
            pltpu.async_copy(g_hbm.at[src_t.at[ci]], rows_v.at[b], sg[b])

        def gather_wait(ci, b):
            pltpu.make_async_copy(g_hbm.at[src_t.at[ci]], rows_v.at[b], sg[b]).wait()

        def scat(ci, b):
            pltpu.async_copy(rows_v.at[b], acc_sh.at[dst_t.at[ci]], ssc, add=True)

        def scat_wait(ci, b):
            pltpu.make_async_copy(rows_v.at[b], acc_sh.at[dst_t.at[ci]], ssc).wait()

        for b in range(NB):
            gather(b, b)

        def fill_zero(r, _):
            for c in range(width // L):
                zero_v[r, pl.ds(c * L, L)] = jnp.zeros((L,), jnp.float32)
            return 0

        lax.fori_loop(0, ZR, fill_zero, 0)

        base = sid * rpt

        def zero_acc(i, _):
            pltpu.sync_copy(zero_v, acc_sh.at[pl.ds(base + i * ZR, ZR)])
            return 0

        lax.fori_loop(0, rpt // ZR, zero_acc, 0)
        plsc.subcore_barrier()

        def pair(pi, _):
            for p in (0, 1):
                gi = 2 * pi + p
                o = p * NB
                oo = (1 - p) * NB
                @pl.when(gi > 0)
                def _():
                    for b in range(NB):
                        scat_wait(NB * (gi - 1) + b, oo + b)

                @pl.when(gi + 1 < ngroups)
                def _():
                    for b in range(NB):
                        gather(NB * (gi + 1) + b, oo + b)

                for b in range(NB):
                    gather_wait(NB * gi + b, o + b)
                    scat(NB * gi + b, o + b)
            return 0

        lax.fori_loop(0, ngroups // 2, pair, 0)

        for b in range(NB):
            scat_wait(NB * (ngroups - 1) + b, NB + b)

        plsc.subcore_barrier()
        pltpu.sync_copy(
            acc_sh.at[pl.ds(base, rpt)], out_hbm.at[cid, pl.ds(base, rpt)]
        )

    return scatter_kernel


def _tc1_body(x_ref, w_ref, d0_ref, d1_ref, g_ref, dinv_ref):
    deg = d0_ref[...] + d1_ref[...] + 1.0
    dinv = lax.rsqrt(jnp.maximum(deg, 1.0))
    h = jnp.dot(x_ref[...], w_ref[...], preferred_element_type=jnp.float32)
    g_ref[...] = h * dinv
    dinv_ref[...] = dinv


def _tc2_body(s0_ref, s1_ref, g_ref, dinv_ref, b_ref, w_ref, out_ref):
    dinv = dinv_ref[...]
    h = dinv * (s0_ref[...] + s1_ref[...] + g_ref[...]) + b_ref[...]
    h = jnp.maximum(h, 0.0)
    out_ref[...] = (
        jnp.dot(h, w_ref[...], preferred_element_type=jnp.float32) * dinv
    )


def _tc3_body(s0_ref, s1_ref, g_ref, dinv_ref, b_ref, out_ref):
    out_ref[...] = (
        dinv_ref[...] * (s0_ref[...] + s1_ref[...] + g_ref[...]) + b_ref[...]
    )


def kernel(x, edge_index, W1, b1, W2, b2):
    N, D = x.shape
    H = W1.shape[1]
    C = W2.shape[1]
    E = edge_index.shape[1]

    n_pad = -(-N // (NS * ZR)) * (NS * ZR)
    Cp = -(-C // L) * L

    nchunks = -(-E // (NW * K_EDGE * 2 * NB)) * (2 * NB)
    Et = NW * K_EDGE * nchunks

    src = edge_index[0]
    dst = edge_index[1]
    if Et > E:
        src = jnp.concatenate([src, jnp.zeros((Et - E,), src.dtype)])
        dst = jnp.concatenate([dst, jnp.full((Et - E,), n_pad - 1, dst.dtype)])
    src2d = src.reshape(NW * nchunks, K_EDGE)
    dst2d = dst.reshape(NW * nchunks, K_EDGE)

    degp = _make_deg_kernel(nchunks, n_pad, K_EDGE)(dst2d)
    d0 = degp[0, :N, 0:1]
    d1 = degp[1, :N, 0:1]

    grid = (N // BM,)
    g1, dinv = pl.pallas_call(
        _tc1_body,
        grid=grid,
        in_specs=[
            pl.BlockSpec((BM, D), lambda i: (i, 0)),
            pl.BlockSpec((D, H), lambda i: (0, 0)),
            pl.BlockSpec((BM, 1), lambda i: (i, 0)),
            pl.BlockSpec((BM, 1), lambda i: (i, 0)),
        ],
        out_specs=[
            pl.BlockSpec((BM, H), lambda i: (i, 0)),
            pl.BlockSpec((BM, 1), lambda i: (i, 0)),
        ],
        out_shape=[
            jax.ShapeDtypeStruct((N, H), jnp.float32),
            jax.ShapeDtypeStruct((N, 1), jnp.float32),
        ],
    )(x, W1, d0, d1)

    s1 = _make_scatter_kernel(H, nchunks, n_pad, K_EDGE)(g1, src2d, dst2d)

    W2p = jnp.pad(W2, ((0, 0), (0, Cp - C)))
    b1r = b1.reshape(1, H)
    g2 = pl.pallas_call(
        _tc2_body,
        grid=grid,
        in_specs=[
            pl.BlockSpec((BM, H), lambda i: (i, 0)),
            pl.BlockSpec((BM, H), lambda i: (i, 0)),
            pl.BlockSpec((BM, H), lambda i: (i, 0)),
            pl.BlockSpec((BM, 1), lambda i: (i, 0)),
            pl.BlockSpec((1, H), lambda i: (0, 0)),
            pl.BlockSpec((H, Cp), lambda i: (0, 0)),
        ],
        out_specs=pl.BlockSpec((BM, Cp), lambda i: (i, 0)),
        out_shape=jax.ShapeDtypeStruct((N, Cp), jnp.float32),
    )(s1[0, :N], s1[1, :N], g1, dinv, b1r, W2p)

    s2 = _make_scatter_kernel(Cp, nchunks, n_pad, K_EDGE)(g2, src2d, dst2d)

    b2r = jnp.pad(b2, (0, Cp - C)).reshape(1, Cp)
    out = pl.pallas_call(
        _tc3_body,
        grid=grid,
        in_specs=[
            pl.BlockSpec((BM, Cp), lambda i: (i, 0)),
            pl.BlockSpec((BM, Cp), lambda i: (i, 0)),
            pl.BlockSpec((BM, Cp), lambda i: (i, 0)),
            pl.BlockSpec((BM, 1), lambda i: (i, 0)),
            pl.BlockSpec((1, Cp), lambda i: (0, 0)),
        ],
        out_specs=pl.BlockSpec((BM, Cp), lambda i: (i, 0)),
        out_shape=jax.ShapeDtypeStruct((N, Cp), jnp.float32),
    )(s2[0, :N], s2[1, :N], g2, dinv, b2r)

    return out[:, :C]


# speedup vs baseline: 18.9263x; 1.0426x over previous
import functools

import jax
import jax.numpy as jnp
from jax import lax
from jax.experimental import pallas as pl
from jax.experimental.pallas import tpu as pltpu
from jax.experimental.pallas import tpu_sc as plsc

NC = 2
NS = 16
L = 16
NW = NC * NS

K_EDGE = 128
NB = 2
ZR = 64
BM = 1000

SPLIT_SCAT = (120, 40)
SPLIT_DEG = (100, 60)
S_CHUNKS = 160


def _mesh():
    return plsc.VectorSubcoreMesh(
        core_axis_name="c", subcore_axis_name="s", num_cores=NC, num_subcores=NS
    )


def _tile_layout(cid, sid, n0, n1):
    is0 = cid == 0
    my_n = jnp.where(is0, n0, n1)
    base = jnp.where(is0, sid * n0, NS * n0 + sid * n1)
    return base, my_n


@functools.lru_cache(None)
def _make_deg_kernel(n0, n1, n_pad, K):
    rpt = n_pad // NS
    QD = 8
    nmax = max(n0, n1)

    @functools.partial(
        pl.kernel,
        out_type=jax.ShapeDtypeStruct((NC, n_pad, L), jnp.float32),
        mesh=_mesh(),
        scratch_types=[
            pltpu.VMEM((nmax, K), jnp.int32),
            pltpu.VMEM((K, L), jnp.float32),
            pltpu.VMEM((ZR, L), jnp.float32),
            pltpu.VMEM_SHARED((n_pad, L), jnp.float32),
            pltpu.SemaphoreType.DMA,
        ],
        compiler_params=pltpu.CompilerParams(use_tc_tiling_on_sc=False),
    )
    def deg_kernel(dst_hbm, out_hbm, dst_t, ones_v, zero_v, acc_sh, ssc):
        cid = lax.axis_index("c")
        sid = lax.axis_index("s")
        row_base, my_n = _tile_layout(cid, sid, n0, n1)

        pltpu.sync_copy(dst_hbm.at[pl.ds(row_base, nmax)], dst_t)

        def fill_ones(r, _):
            ones_v[r, :] = jnp.full((L,), 1.0, jnp.float32)
            return 0

        lax.fori_loop(0, K, fill_ones, 0)

        def fill_zero(r, _):
            zero_v[r, :] = jnp.zeros((L,), jnp.float32)
            return 0

        lax.fori_loop(0, ZR, fill_zero, 0)

        base = sid * rpt

        def zero_acc(i, _):
            pltpu.sync_copy(zero_v, acc_sh.at[pl.ds(base + i * ZR, ZR)])
            return 0

        lax.fori_loop(0, rpt // ZR, zero_acc, 0)
        plsc.subcore_barrier()

        for j in range(QD):
            pltpu.async_copy(ones_v, acc_sh.at[dst_t.at[j]], ssc, add=True)

        def body(ci, _):
            pltpu.make_async_copy(ones_v, acc_sh.at[dst_t.at[ci]], ssc).wait()
            pltpu.async_copy(ones_v, acc_sh.at[dst_t.at[ci + QD]], ssc, add=True)
            return 0

        lax.fori_loop(0, my_n - QD, body, 0)

        def drain(j, _):
            pltpu.make_async_copy(
                ones_v, acc_sh.at[dst_t.at[my_n - QD + j]], ssc
            ).wait()
            return 0

        lax.fori_loop(0, QD, drain, 0)
        plsc.subcore_barrier()
        pltpu.sync_copy(
            acc_sh.at[pl.ds(base, rpt)], out_hbm.at[cid, pl.ds(base, rpt)]
        )

    return deg_kernel


@functools.lru_cache(None)
def _make_scatter_kernel(width, n0, n1, n_pad, K):
    rpt = n_pad // NS
    nmax = max(n0, n1)

    @functools.partial(
        pl.kernel,
        out_type=jax.ShapeDtypeStruct((NC, n_pad, width), jnp.float32),
        mesh=_mesh(),
        scratch_types=[
            pltpu.VMEM((nmax, K), jnp.int32),
            pltpu.VMEM((nmax, K), jnp.int32),
            pltpu.VMEM((2 * NB, K, width), jnp.float32),
            pltpu.VMEM((ZR, width), jnp.float32),
            pltpu.VMEM_SHARED((n_pad, width), jnp.float32),
        ]
        + [pltpu.SemaphoreType.DMA] * (2 * NB)
        + [pltpu.SemaphoreType.DMA],
        compiler_params=pltpu.CompilerParams(use_tc_tiling_on_sc=False),
    )
    def scatter_kernel(
        g_hbm, src_hbm, dst_hbm, out_hbm, src_t, dst_t, rows_v, zero_v, acc_sh, *sems
    ):
        sg = sems[: 2 * NB]
        ssc = sems[2 * NB]
        cid = lax.axis_index("c")
        sid = lax.axis_index("s")
        row_base, my_n = _tile_layout(cid, sid, n0, n1)
        my_groups = my_n // NB

        pltpu.sync_copy(src_hbm.at[pl.ds(row_base, nmax)], src_t)
        pltpu.sync_copy(dst_hbm.at[pl.ds(row_base, nmax)], dst_t)

        def gather(ci, b):
            pltpu.async_copy(g_hbm.at[src_t.at[ci]], rows_v.at[b], sg[b])

        def gather_wait(ci, b):
            pltpu.make_async_copy(g_hbm.at[src_t.at[ci]], rows_v.at[b], sg[b]).wait()

        def scat(ci, b):
            pltpu.async_copy(rows_v.at[b], acc_sh.at[dst_t.at[ci]], ssc, add=True)

        def scat_wait(ci, b):
            pltpu.make_async_copy(rows_v.at[b], acc_sh.at[dst_t.at[ci]], ssc).wait()

        for b in range(NB):
            gather(b, b)

        def fill_zero(r, _):
            for c in range(width // L):
                zero_v[r, pl.ds(c * L, L)] = jnp.zeros((L,), jnp.float32)
            return 0

        lax.fori_loop(0, ZR, fill_zero, 0)

        base = sid * rpt

        def zero_acc(i, _):
            pltpu.sync_copy(zero_v, acc_sh.at[pl.ds(base + i * ZR, ZR)])
            return 0

        lax.fori_loop(0, rpt // ZR, zero_acc, 0)
        plsc.subcore_barrier()

        def pair(pi, _):
            for p in (0, 1):
                gi = 2 * pi + p
                o = p * NB
                oo = (1 - p) * NB
                @pl.when(gi > 0)
                def _():
                    for b in range(NB):
                        scat_wait(NB * (gi - 1) + b, oo + b)

                @pl.when(gi + 1 < my_groups)
                def _():
                    for b in range(NB):
                        gather(NB * (gi + 1) + b, oo + b)

                for b in range(NB):
                    gather_wait(NB * gi + b, o + b)
                    scat(NB * gi + b, o + b)
            return 0

        lax.fori_loop(0, my_groups // 2, pair, 0)

        for b in range(NB):
            scat_wait(NB * (my_groups - 1) + b, NB + b)

        plsc.subcore_barrier()
        pltpu.sync_copy(
            acc_sh.at[pl.ds(base, rpt)], out_hbm.at[cid, pl.ds(base, rpt)]
        )

    return scatter_kernel


def _tc1_body(x_ref, w_ref, d0_ref, d1_ref, g_ref, dinv_ref):
    deg = d0_ref[...] + d1_ref[...] + 1.0
    dinv = lax.rsqrt(jnp.maximum(deg, 1.0))
    h = jnp.dot(x_ref[...], w_ref[...], preferred_element_type=jnp.float32)
    g_ref[...] = h * dinv
    dinv_ref[...] = dinv


def _tc2_body(s0_ref, s1_ref, g_ref, dinv_ref, b_ref, w_ref, out_ref):
    dinv = dinv_ref[...]
    h = dinv * (s0_ref[...] + s1_ref[...] + g_ref[...]) + b_ref[...]
    h = jnp.maximum(h, 0.0)
    out_ref[...] = (
        jnp.dot(h, w_ref[...], preferred_element_type=jnp.float32) * dinv
    )


def _tc3_body(s0_ref, s1_ref, g_ref, dinv_ref, b_ref, out_ref):
    out_ref[...] = (
        dinv_ref[...] * (s0_ref[...] + s1_ref[...] + g_ref[...]) + b_ref[...]
    )


def kernel(x, edge_index, W1, b1, W2, b2):
    N, D = x.shape
    H = W1.shape[1]
    C = W2.shape[1]
    E = edge_index.shape[1]

    n_pad = -(-N // (NS * ZR)) * (NS * ZR)
    Cp = -(-C // L) * L

    n_rows = NS * S_CHUNKS
    pad_rows = max(max(SPLIT_SCAT), max(SPLIT_DEG))
    Et = (n_rows + pad_rows) * K_EDGE
    assert NS * S_CHUNKS * K_EDGE >= E

    src = edge_index[0]
    dst = edge_index[1]
    src = jnp.concatenate([src, jnp.zeros((Et - E,), src.dtype)])
    dst = jnp.concatenate([dst, jnp.full((Et - E,), n_pad - 1, dst.dtype)])
    src2d = src.reshape(n_rows + pad_rows, K_EDGE)
    dst2d = dst.reshape(n_rows + pad_rows, K_EDGE)

    degp = _make_deg_kernel(*SPLIT_DEG, n_pad, K_EDGE)(dst2d)
    d0 = degp[0, :N, 0:1]
    d1 = degp[1, :N, 0:1]

    grid = (N // BM,)
    g1, dinv = pl.pallas_call(
        _tc1_body,
        grid=grid,
        in_specs=[
            pl.BlockSpec((BM, D), lambda i: (i, 0)),
            pl.BlockSpec((D, H), lambda i: (0, 0)),
            pl.BlockSpec((BM, 1), lambda i: (i, 0)),
            pl.BlockSpec((BM, 1), lambda i: (i, 0)),
        ],
        out_specs=[
            pl.BlockSpec((BM, H), lambda i: (i, 0)),
            pl.BlockSpec((BM, 1), lambda i: (i, 0)),
        ],
        out_shape=[
            jax.ShapeDtypeStruct((N, H), jnp.float32),
            jax.ShapeDtypeStruct((N, 1), jnp.float32),
        ],
    )(x, W1, d0, d1)

    s1 = _make_scatter_kernel(H, *SPLIT_SCAT, n_pad, K_EDGE)(g1, src2d, dst2d)

    W2p = jnp.pad(W2, ((0, 0), (0, Cp - C)))
    b1r = b1.reshape(1, H)
    g2 = pl.pallas_call(
        _tc2_body,
        grid=grid,
        in_specs=[
            pl.BlockSpec((BM, H), lambda i: (i, 0)),
            pl.BlockSpec((BM, H), lambda i: (i, 0)),
            pl.BlockSpec((BM, H), lambda i: (i, 0)),
            pl.BlockSpec((BM, 1), lambda i: (i, 0)),
            pl.BlockSpec((1, H), lambda i: (0, 0)),
            pl.BlockSpec((H, Cp), lambda i: (0, 0)),
        ],
        out_specs=pl.BlockSpec((BM, Cp), lambda i: (i, 0)),
        out_shape=jax.ShapeDtypeStruct((N, Cp), jnp.float32),
    )(s1[0, :N], s1[1, :N], g1, dinv, b1r, W2p)

    s2 = _make_scatter_kernel(Cp, *SPLIT_SCAT, n_pad, K_EDGE)(g2, src2d, dst2d)

    b2r = jnp.pad(b2, (0, Cp - C)).reshape(1, Cp)
    out = pl.pallas_call(
        _tc3_body,
        grid=grid,
        in_specs=[
            pl.BlockSpec((BM, Cp), lambda i: (i, 0)),
            pl.BlockSpec((BM, Cp), lambda i: (i, 0)),
            pl.BlockSpec((BM, Cp), lambda i: (i, 0)),
            pl.BlockSpec((BM, 1), lambda i: (i, 0)),
            pl.BlockSpec((1, Cp), lambda i: (0, 0)),
        ],
        out_specs=pl.BlockSpec((BM, Cp), lambda i: (i, 0)),
        out_shape=jax.ShapeDtypeStruct((N, Cp), jnp.float32),
    )(s2[0, :N], s2[1, :N], g2, dinv, b2r)

    return out[:, :C]


# speedup vs baseline: 20.8349x; 1.1008x over previous
import functools

import jax
import jax.numpy as jnp
from jax import lax
from jax.experimental import pallas as pl
from jax.experimental.pallas import tpu as pltpu
from jax.experimental.pallas import tpu_sc as plsc

NC = 2
NS = 16
L = 16
NW = NC * NS

K_EDGE = 128
NB = 2
ZR = 64
BM = 1000

SPLIT_SCAT = (152, 8)
SPLIT_DEG = (144, 16)
S_CHUNKS = 160


def _mesh():
    return plsc.VectorSubcoreMesh(
        core_axis_name="c", subcore_axis_name="s", num_cores=NC, num_subcores=NS
    )


def _tile_layout(cid, sid, n0, n1):
    is0 = cid == 0
    my_n = jnp.where(is0, n0, n1)
    base = jnp.where(is0, sid * n0, NS * n0 + sid * n1)
    return base, my_n


@functools.lru_cache(None)
def _make_deg_kernel(n0, n1, n_pad, K):
    rpt = n_pad // NS
    QD = 8
    nmax = max(n0, n1)

    @functools.partial(
        pl.kernel,
        out_type=jax.ShapeDtypeStruct((NC, n_pad, L), jnp.float32),
        mesh=_mesh(),
        scratch_types=[
            pltpu.VMEM((nmax, K), jnp.int32),
            pltpu.VMEM((K, L), jnp.float32),
            pltpu.VMEM((ZR, L), jnp.float32),
            pltpu.VMEM_SHARED((n_pad, L), jnp.float32),
            pltpu.SemaphoreType.DMA,
        ],
        compiler_params=pltpu.CompilerParams(use_tc_tiling_on_sc=False),
    )
    def deg_kernel(dst_hbm, out_hbm, dst_t, ones_v, zero_v, acc_sh, ssc):
        cid = lax.axis_index("c")
        sid = lax.axis_index("s")
        row_base, my_n = _tile_layout(cid, sid, n0, n1)

        pltpu.sync_copy(dst_hbm.at[pl.ds(row_base, nmax)], dst_t)

        def fill_ones(r, _):
            ones_v[r, :] = jnp.full((L,), 1.0, jnp.float32)
            return 0

        lax.fori_loop(0, K, fill_ones, 0)

        def fill_zero(r, _):
            zero_v[r, :] = jnp.zeros((L,), jnp.float32)
            return 0

        lax.fori_loop(0, ZR, fill_zero, 0)

        base = sid * rpt

        def zero_acc(i, _):
            pltpu.sync_copy(zero_v, acc_sh.at[pl.ds(base + i * ZR, ZR)])
            return 0

        lax.fori_loop(0, rpt // ZR, zero_acc, 0)
        plsc.subcore_barrier()

        for j in range(QD):
            pltpu.async_copy(ones_v, acc_sh.at[dst_t.at[j]], ssc, add=True)

        def body(ci, _):
            pltpu.make_async_copy(ones_v, acc_sh.at[dst_t.at[ci]], ssc).wait()
            pltpu.async_copy(ones_v, acc_sh.at[dst_t.at[ci + QD]], ssc, add=True)
            return 0

        lax.fori_loop(0, my_n - QD, body, 0)

        def drain(j, _):
            pltpu.make_async_copy(
                ones_v, acc_sh.at[dst_t.at[my_n - QD + j]], ssc
            ).wait()
            return 0

        lax.fori_loop(0, QD, drain, 0)
        plsc.subcore_barrier()
        pltpu.sync_copy(
            acc_sh.at[pl.ds(base, rpt)], out_hbm.at[cid, pl.ds(base, rpt)]
        )

    return deg_kernel


@functools.lru_cache(None)
def _make_scatter_kernel(width, n0, n1, n_pad, K):
    rpt = n_pad // NS
    nmax = max(n0, n1)

    @functools.partial(
        pl.kernel,
        out_type=jax.ShapeDtypeStruct((NC, n_pad, width), jnp.float32),
        mesh=_mesh(),
        scratch_types=[
            pltpu.VMEM((nmax, K), jnp.int32),
            pltpu.VMEM((nmax, K), jnp.int32),
            pltpu.VMEM((2 * NB, K, width), jnp.float32),
            pltpu.VMEM((ZR, width), jnp.float32),
            pltpu.VMEM_SHARED((n_pad, width), jnp.float32),
        ]
        + [pltpu.SemaphoreType.DMA] * (2 * NB)
        + [pltpu.SemaphoreType.DMA],
        compiler_params=pltpu.CompilerParams(use_tc_tiling_on_sc=False),
    )
    def scatter_kernel(
        g_hbm, src_hbm, dst_hbm, out_hbm, src_t, dst_t, rows_v, zero_v, acc_sh, *sems
    ):
        sg = sems[: 2 * NB]
        ssc = sems[2 * NB]
        cid = lax.axis_index("c")
        sid = lax.axis_index("s")
        row_base, my_n = _tile_layout(cid, sid, n0, n1)
        my_groups = my_n // NB

        pltpu.sync_copy(src_hbm.at[pl.ds(row_base, nmax)], src_t)
        pltpu.sync_copy(dst_hbm.at[pl.ds(row_base, nmax)], dst_t)

        def gather(ci, b):
            pltpu.async_copy(g_hbm.at[src_t.at[ci]], rows_v.at[b], sg[b])

        def gather_wait(ci, b):
            pltpu.make_async_copy(g_hbm.at[src_t.at[ci]], rows_v.at[b], sg[b]).wait()

        def scat(ci, b):
            pltpu.async_copy(rows_v.at[b], acc_sh.at[dst_t.at[ci]], ssc, add=True)

        def scat_wait(ci, b):
            pltpu.make_async_copy(rows_v.at[b], acc_sh.at[dst_t.at[ci]], ssc).wait()

        for b in range(NB):
            gather(b, b)

        def fill_zero(r, _):
            for c in range(width // L):
                zero_v[r, pl.ds(c * L, L)] = jnp.zeros((L,), jnp.float32)
            return 0

        lax.fori_loop(0, ZR, fill_zero, 0)

        base = sid * rpt

        def zero_acc(i, _):
            pltpu.sync_copy(zero_v, acc_sh.at[pl.ds(base + i * ZR, ZR)])
            return 0

        lax.fori_loop(0, rpt // ZR, zero_acc, 0)
        plsc.subcore_barrier()

        def pair(pi, _):
            for p in (0, 1):
                gi = 2 * pi + p
                o = p * NB
                oo = (1 - p) * NB
                @pl.when(gi > 0)
                def _():
                    for b in range(NB):
                        scat_wait(NB * (gi - 1) + b, oo + b)

                @pl.when(gi + 1 < my_groups)
                def _():
                    for b in range(NB):
                        gather(NB * (gi + 1) + b, oo + b)

                for b in range(NB):
                    gather_wait(NB * gi + b, o + b)
                    scat(NB * gi + b, o + b)
            return 0

        lax.fori_loop(0, my_groups // 2, pair, 0)

        for b in range(NB):
            scat_wait(NB * (my_groups - 1) + b, NB + b)

        plsc.subcore_barrier()
        pltpu.sync_copy(
            acc_sh.at[pl.ds(base, rpt)], out_hbm.at[cid, pl.ds(base, rpt)]
        )

    return scatter_kernel


def _tc1_body(x_ref, w_ref, d0_ref, d1_ref, g_ref, dinv_ref):
    deg = d0_ref[...] + d1_ref[...] + 1.0
    dinv = lax.rsqrt(jnp.maximum(deg, 1.0))
    h = jnp.dot(x_ref[...], w_ref[...], preferred_element_type=jnp.float32)
    g_ref[...] = h * dinv
    dinv_ref[...] = dinv


def _tc2_body(s0_ref, s1_ref, g_ref, dinv_ref, b_ref, w_ref, out_ref):
    dinv = dinv_ref[...]
    h = dinv * (s0_ref[...] + s1_ref[...] + g_ref[...]) + b_ref[...]
    h = jnp.maximum(h, 0.0)
    out_ref[...] = (
        jnp.dot(h, w_ref[...], preferred_element_type=jnp.float32) * dinv
    )


def _tc3_body(s0_ref, s1_ref, g_ref, dinv_ref, b_ref, out_ref):
    out_ref[...] = (
        dinv_ref[...] * (s0_ref[...] + s1_ref[...] + g_ref[...]) + b_ref[...]
    )


def kernel(x, edge_index, W1, b1, W2, b2):
    N, D = x.shape
    H = W1.shape[1]
    C = W2.shape[1]
    E = edge_index.shape[1]

    n_pad = -(-N // (NS * ZR)) * (NS * ZR)
    Cp = -(-C // L) * L

    n_rows = NS * S_CHUNKS
    pad_rows = max(max(SPLIT_SCAT), max(SPLIT_DEG))
    Et = (n_rows + pad_rows) * K_EDGE
    assert NS * S_CHUNKS * K_EDGE >= E

    src = edge_index[0]
    dst = edge_index[1]
    src = jnp.concatenate([src, jnp.zeros((Et - E,), src.dtype)])
    dst = jnp.concatenate([dst, jnp.full((Et - E,), n_pad - 1, dst.dtype)])
    src2d = src.reshape(n_rows + pad_rows, K_EDGE)
    dst2d = dst.reshape(n_rows + pad_rows, K_EDGE)

    degp = _make_deg_kernel(*SPLIT_DEG, n_pad, K_EDGE)(dst2d)
    d0 = degp[0, :N, 0:1]
    d1 = degp[1, :N, 0:1]

    grid = (N // BM,)
    g1, dinv = pl.pallas_call(
        _tc1_body,
        grid=grid,
        in_specs=[
            pl.BlockSpec((BM, D), lambda i: (i, 0)),
            pl.BlockSpec((D, H), lambda i: (0, 0)),
            pl.BlockSpec((BM, 1), lambda i: (i, 0)),
            pl.BlockSpec((BM, 1), lambda i: (i, 0)),
        ],
        out_specs=[
            pl.BlockSpec((BM, H), lambda i: (i, 0)),
            pl.BlockSpec((BM, 1), lambda i: (i, 0)),
        ],
        out_shape=[
            jax.ShapeDtypeStruct((N, H), jnp.float32),
            jax.ShapeDtypeStruct((N, 1), jnp.float32),
        ],
    )(x, W1, d0, d1)

    s1 = _make_scatter_kernel(H, *SPLIT_SCAT, n_pad, K_EDGE)(g1, src2d, dst2d)

    W2p = jnp.pad(W2, ((0, 0), (0, Cp - C)))
    b1r = b1.reshape(1, H)
    g2 = pl.pallas_call(
        _tc2_body,
        grid=grid,
        in_specs=[
            pl.BlockSpec((BM, H), lambda i: (i, 0)),
            pl.BlockSpec((BM, H), lambda i: (i, 0)),
            pl.BlockSpec((BM, H), lambda i: (i, 0)),
            pl.BlockSpec((BM, 1), lambda i: (i, 0)),
            pl.BlockSpec((1, H), lambda i: (0, 0)),
            pl.BlockSpec((H, Cp), lambda i: (0, 0)),
        ],
        out_specs=pl.BlockSpec((BM, Cp), lambda i: (i, 0)),
        out_shape=jax.ShapeDtypeStruct((N, Cp), jnp.float32),
    )(s1[0, :N], s1[1, :N], g1, dinv, b1r, W2p)

    s2 = _make_scatter_kernel(Cp, *SPLIT_SCAT, n_pad, K_EDGE)(g2, src2d, dst2d)

    b2r = jnp.pad(b2, (0, Cp - C)).reshape(1, Cp)
    out = pl.pallas_call(
        _tc3_body,
        grid=grid,
        in_specs=[
            pl.BlockSpec((BM, Cp), lambda i: (i, 0)),
            pl.BlockSpec((BM, Cp), lambda i: (i, 0)),
            pl.BlockSpec((BM, Cp), lambda i: (i, 0)),
            pl.BlockSpec((BM, 1), lambda i: (i, 0)),
            pl.BlockSpec((1, Cp), lambda i: (0, 0)),
        ],
        out_specs=pl.BlockSpec((BM, Cp), lambda i: (i, 0)),
        out_shape=jax.ShapeDtypeStruct((N, Cp), jnp.float32),
    )(s2[0, :N], s2[1, :N], g2, dinv, b2r)

    return out[:, :C]
